# Initial kernel scaffold; baseline (speedup 1.0000x reference)
import jax
import jax.numpy as jnp
from jax.experimental import pallas as pl


def kernel(b1c1_w9, b1c1_b, b1c1_gamma, b1c1_beta, b1c2_w9, b1c2_b, b1c2_gamma, b1c2_beta, b2c1_w9, b2c1_b, b2c1_gamma, b2c1_beta, b2c2_w9, b2c2_b, b2c2_gamma, b2c2_beta, b3c1_w9, b3c1_b, b3c1_gamma, b3c1_beta, b3c2_w9, b3c2_b, b3c2_gamma, b3c2_beta, fc1_w, fc1_b, fc2_w, fc2_b, x, m1, m2):
    raise NotImplementedError("write your pallas kernel here")


# speedup vs baseline: 1.8085x; 1.8085x over previous
import functools

import jax
import jax.numpy as jnp
from jax.experimental import pallas as pl
from jax.experimental.pallas import tpu as pltpu

_EPS = 1e-5
_VMEM_LIMIT = 48 * 1024 * 1024


def _tile_stats(st_ref, y):
    st_ref[0, 0:1, :] = jnp.sum(y, axis=0, keepdims=True)
    st_ref[0, 1:2, :] = jnp.sum(y * y, axis=0, keepdims=True)


def _conv1_kernel(xp_ref, w_ref, b_ref, y_ref, st_ref, col_ref, *, BT):
    H = W = 32
    R = H * W
    for k in range(9):
        ky, kx = divmod(k, 3)
        col_ref[:, 8 * k:8 * k + 8] = (
            xp_ref[:, ky:ky + H, kx:kx + W, :].reshape(BT * R, 8))
    acc = jnp.dot(col_ref[...], w_ref[...],
                  preferred_element_type=jnp.float32)
    y = jnp.maximum(acc + b_ref[...], 0.0)
    y_ref[...] = y.astype(jnp.bfloat16)
    _tile_stats(st_ref, y)


def _bn_conv_kernel(y_ref, st_in_ref, g_ref, be_ref, w_ref, b_ref,
                    o_ref, st_ref, xp_ref, *, BT, Hi, pool, Cin, Cout, Np):
    sums = jnp.sum(st_in_ref[...], axis=0)
    mean = sums[0:1, :] * (1.0 / Np)
    var = jnp.maximum(sums[1:2, :] * (1.0 / Np) - mean * mean, 0.0)
    scale = g_ref[...] * jax.lax.rsqrt(var + _EPS)
    shift = be_ref[...] - mean * scale

    z = y_ref[...].astype(jnp.float32) * scale + shift
    H = Hi // 2 if pool else Hi
    if pool:
        z = z.reshape(BT * Hi * H, 2, Cin)
        z = jnp.maximum(z[:, 0, :], z[:, 1, :])
        z = z.reshape(BT * H, 2, H, Cin)
        z = jnp.maximum(z[:, 0], z[:, 1])
    x = z.astype(jnp.bfloat16).reshape(BT, H, H, Cin)

    xp_ref[...] = jnp.zeros_like(xp_ref)
    xp_ref[:, 1:H + 1, 1:H + 1, :] = x

    R = H * H
    acc = jnp.zeros((BT * R, Cout), jnp.float32)
    for k in range(9):
        ky, kx = divmod(k, 3)
        tap = xp_ref[:, ky:ky + H, kx:kx + H, :].reshape(BT * R, Cin)
        acc = acc + jnp.dot(tap, w_ref[k],
                            preferred_element_type=jnp.float32)
    y = jnp.maximum(acc + b_ref[...], 0.0)
    o_ref[...] = y.astype(jnp.bfloat16)
    _tile_stats(st_ref, y)


def _head_kernel(y_ref, st_in_ref, g_ref, be_ref, m1_ref, w1_ref, b1_ref,
                 m2_ref, w2_ref, b2_ref, o_ref, *, BT, Np):
    sums = jnp.sum(st_in_ref[...], axis=0)
    mean = sums[0:1, :] * (1.0 / Np)
    var = jnp.maximum(sums[1:2, :] * (1.0 / Np) - mean * mean, 0.0)
    scale = g_ref[...] * jax.lax.rsqrt(var + _EPS)
    shift = be_ref[...] - mean * scale

    z = y_ref[...].astype(jnp.float32) * scale + shift
    z = z.reshape(BT * 8 * 4, 2, 128)
    z = jnp.maximum(z[:, 0, :], z[:, 1, :])
    z = z.reshape(BT * 4, 2, 4, 128)
    z = jnp.maximum(z[:, 0], z[:, 1])
    xf = z.astype(jnp.bfloat16).reshape(BT, 2048)

    xf = xf * m1_ref[...]
    h = jnp.dot(xf, w1_ref[...], preferred_element_type=jnp.float32)
    h = jnp.maximum(h + b1_ref[...], 0.0)
    h = h.astype(jnp.bfloat16) * m2_ref[...]
    y = jnp.dot(h, w2_ref[...], preferred_element_type=jnp.float32)
    o_ref[...] = y + b2_ref[...]


def _conv_stage(yprev, st_prev, gamma, beta, w9, b, *,
                BT, Hi, pool, Cin, Cout):
    H = Hi // 2 if pool else Hi
    R = H * H
    Rp = Hi * Hi
    _B = yprev.shape[0] // Rp
    T = _B // BT
    Tp = st_prev.shape[0]
    y, st = pl.pallas_call(
        functools.partial(_bn_conv_kernel, BT=BT, Hi=Hi, pool=pool,
                          Cin=Cin, Cout=Cout, Np=_B * Rp),
        out_shape=(jax.ShapeDtypeStruct((_B * R, Cout), jnp.bfloat16),
                   jax.ShapeDtypeStruct((T, 2, Cout), jnp.float32)),
        grid=(T,),
        in_specs=[
            pl.BlockSpec((BT * Rp, Cin), lambda t: (t, 0)),
            pl.BlockSpec((Tp, 2, Cin), lambda t: (0, 0, 0)),
            pl.BlockSpec((1, Cin), lambda t: (0, 0)),
            pl.BlockSpec((1, Cin), lambda t: (0, 0)),
            pl.BlockSpec((9, Cin, Cout), lambda t: (0, 0, 0)),
            pl.BlockSpec((1, Cout), lambda t: (0, 0)),
        ],
        out_specs=(
            pl.BlockSpec((BT * R, Cout), lambda t: (t, 0)),
            pl.BlockSpec((1, 2, Cout), lambda t: (t, 0, 0)),
        ),
        scratch_shapes=[pltpu.VMEM((BT, H + 2, H + 2, Cin), jnp.bfloat16)],
        compiler_params=pltpu.CompilerParams(
            dimension_semantics=("parallel",),
            vmem_limit_bytes=_VMEM_LIMIT),
    )(yprev, st_prev, gamma.reshape(1, Cin), beta.reshape(1, Cin), w9, b)
    return y, st


def kernel(b1c1_w9, b1c1_b, b1c1_gamma, b1c1_beta,
           b1c2_w9, b1c2_b, b1c2_gamma, b1c2_beta,
           b2c1_w9, b2c1_b, b2c1_gamma, b2c1_beta,
           b2c2_w9, b2c2_b, b2c2_gamma, b2c2_beta,
           b3c1_w9, b3c1_b, b3c1_gamma, b3c1_beta,
           b3c2_w9, b3c2_b, b3c2_gamma, b3c2_beta,
           fc1_w, fc1_b, fc2_w, fc2_b,
           x, m1, m2):
    B = x.shape[0]
    xh = jnp.transpose(x, (0, 2, 3, 1)).astype(jnp.bfloat16)
    xpad = jnp.pad(xh, ((0, 0), (1, 1), (1, 1), (0, 5)))
    w72 = jnp.zeros((9, 8, 32), jnp.bfloat16).at[:, :3, :].set(
        b1c1_w9).reshape(72, 32)

    BT1 = 8
    T1 = B // BT1
    y1, st1 = pl.pallas_call(
        functools.partial(_conv1_kernel, BT=BT1),
        out_shape=(jax.ShapeDtypeStruct((B * 1024, 32), jnp.bfloat16),
                   jax.ShapeDtypeStruct((T1, 2, 32), jnp.float32)),
        grid=(T1,),
        in_specs=[
            pl.BlockSpec((BT1, 34, 34, 8), lambda t: (t, 0, 0, 0)),
            pl.BlockSpec((72, 32), lambda t: (0, 0)),
            pl.BlockSpec((1, 32), lambda t: (0, 0)),
        ],
        out_specs=(
            pl.BlockSpec((BT1 * 1024, 32), lambda t: (t, 0)),
            pl.BlockSpec((1, 2, 32), lambda t: (t, 0, 0)),
        ),
        scratch_shapes=[pltpu.VMEM((BT1 * 1024, 72), jnp.bfloat16)],
        compiler_params=pltpu.CompilerParams(
            dimension_semantics=("parallel",),
            vmem_limit_bytes=_VMEM_LIMIT),
    )(xpad, w72, b1c1_b)

    y2, st2 = _conv_stage(y1, st1, b1c1_gamma, b1c1_beta, b1c2_w9, b1c2_b,
                          BT=8, Hi=32, pool=False, Cin=32, Cout=32)
    y3, st3 = _conv_stage(y2, st2, b1c2_gamma, b1c2_beta, b2c1_w9, b2c1_b,
                          BT=8, Hi=32, pool=True, Cin=32, Cout=64)
    y4, st4 = _conv_stage(y3, st3, b2c1_gamma, b2c1_beta, b2c2_w9, b2c2_b,
                          BT=16, Hi=16, pool=False, Cin=64, Cout=64)
    y5, st5 = _conv_stage(y4, st4, b2c2_gamma, b2c2_beta, b3c1_w9, b3c1_b,
                          BT=16, Hi=16, pool=True, Cin=64, Cout=128)
    y6, st6 = _conv_stage(y5, st5, b3c1_gamma, b3c1_beta, b3c2_w9, b3c2_b,
                          BT=32, Hi=8, pool=False, Cin=128, Cout=128)

    BT7 = 128
    T6 = st6.shape[0]
    Dpad = fc2_w.shape[1]
    logits = pl.pallas_call(
        functools.partial(_head_kernel, BT=BT7, Np=B * 64),
        out_shape=jax.ShapeDtypeStruct((B, Dpad), jnp.float32),
        grid=(B // BT7,),
        in_specs=[
            pl.BlockSpec((BT7 * 64, 128), lambda t: (t, 0)),
            pl.BlockSpec((T6, 2, 128), lambda t: (0, 0, 0)),
            pl.BlockSpec((1, 128), lambda t: (0, 0)),
            pl.BlockSpec((1, 128), lambda t: (0, 0)),
            pl.BlockSpec((BT7, 2048), lambda t: (t, 0)),
            pl.BlockSpec((2048, 1024), lambda t: (0, 0)),
            pl.BlockSpec((1, 1024), lambda t: (0, 0)),
            pl.BlockSpec((BT7, 1024), lambda t: (t, 0)),
            pl.BlockSpec((1024, Dpad), lambda t: (0, 0)),
            pl.BlockSpec((1, Dpad), lambda t: (0, 0)),
        ],
        out_specs=pl.BlockSpec((BT7, Dpad), lambda t: (t, 0)),
        compiler_params=pltpu.CompilerParams(
            dimension_semantics=("parallel",),
            vmem_limit_bytes=_VMEM_LIMIT),
    )(y6, st6, b3c2_gamma.reshape(1, 128), b3c2_beta.reshape(1, 128),
      m1, fc1_w, fc1_b, m2, fc2_w, fc2_b)
    return logits[:, :10]


# speedup vs baseline: 2.9238x; 1.6167x over previous
import functools

import jax
import jax.numpy as jnp
from jax.experimental import pallas as pl
from jax.experimental.pallas import tpu as pltpu

_EPS = 1e-5
_VMEM_LIMIT = 48 * 1024 * 1024


def _tile_stats(st_ref, y):
    st_ref[0, 0:1, :] = jnp.sum(y, axis=0, keepdims=True)
    st_ref[0, 1:2, :] = jnp.sum(y * y, axis=0, keepdims=True)


def _conv1_kernel(x_ref, w_ref, b_ref, y_ref, st_ref, xp_ref, col_ref, *,
                  BT):
    H = W = 32
    R = H * W
    xt = jnp.transpose(x_ref[...].reshape(BT, 3, R), (0, 2, 1))
    xi = xt.astype(jnp.bfloat16).reshape(BT, H, W, 3)
    xp_ref[...] = jnp.zeros_like(xp_ref)
    xp_ref[:, 1:H + 1, 1:W + 1, :3] = xi
    for k in range(9):
        ky, kx = divmod(k, 3)
        col_ref[:, 8 * k:8 * k + 8] = (
            xp_ref[:, ky:ky + H, kx:kx + W, :].reshape(BT * R, 8))
    acc = jnp.dot(col_ref[...], w_ref[...],
                  preferred_element_type=jnp.float32)
    y = jnp.maximum(acc + b_ref[...], 0.0)
    y_ref[...] = y.astype(jnp.bfloat16)
    _tile_stats(st_ref, y)


def _bn_conv_kernel(y_ref, st_in_ref, g_ref, be_ref, w_ref, b_ref,
                    o_ref, st_ref, xp_ref, col_ref=None, *, BT, Hi, pool,
                    Cin, Cout, Np):
    sums = jnp.sum(st_in_ref[...], axis=0)
    mean = sums[0:1, :] * (1.0 / Np)
    var = jnp.maximum(sums[1:2, :] * (1.0 / Np) - mean * mean, 0.0)
    scale = g_ref[...] * jax.lax.rsqrt(var + _EPS)
    shift = be_ref[...] - mean * scale

    z = y_ref[...].astype(jnp.float32) * scale + shift
    H = Hi // 2 if pool else Hi
    if pool:
        z = z.reshape(BT * Hi * H, 2, Cin)
        z = jnp.maximum(z[:, 0, :], z[:, 1, :])
        z = z.reshape(BT * H, 2, H, Cin)
        z = jnp.maximum(z[:, 0], z[:, 1])
    x = z.astype(jnp.bfloat16).reshape(BT, H, H, Cin)

    xp_ref[...] = jnp.zeros_like(xp_ref)
    xp_ref[:, 1:H + 1, 1:H + 1, :] = x

    R = H * H
    if col_ref is not None:
        for k in range(9):
            ky, kx = divmod(k, 3)
            col_ref[:, Cin * k:Cin * (k + 1)] = (
                xp_ref[:, ky:ky + H, kx:kx + H, :].reshape(BT * R, Cin))
        acc = jnp.dot(col_ref[...], w_ref[...],
                      preferred_element_type=jnp.float32)
    else:
        acc = jnp.zeros((BT * R, Cout), jnp.float32)
        for k in range(9):
            ky, kx = divmod(k, 3)
            tap = xp_ref[:, ky:ky + H, kx:kx + H, :].reshape(BT * R, Cin)
            acc = acc + jnp.dot(tap, w_ref[k],
                                preferred_element_type=jnp.float32)
    y = jnp.maximum(acc + b_ref[...], 0.0)
    o_ref[...] = y.astype(jnp.bfloat16)
    _tile_stats(st_ref, y)


def _head_kernel(y_ref, st_in_ref, g_ref, be_ref, m1_ref, w1_ref, b1_ref,
                 m2_ref, w2_ref, b2_ref, o_ref, *, BT, Np):
    sums = jnp.sum(st_in_ref[...], axis=0)
    mean = sums[0:1, :] * (1.0 / Np)
    var = jnp.maximum(sums[1:2, :] * (1.0 / Np) - mean * mean, 0.0)
    scale = g_ref[...] * jax.lax.rsqrt(var + _EPS)
    shift = be_ref[...] - mean * scale

    z = y_ref[...].astype(jnp.float32) * scale + shift
    z = z.reshape(BT * 8 * 4, 2, 128)
    z = jnp.maximum(z[:, 0, :], z[:, 1, :])
    z = z.reshape(BT * 4, 2, 4, 128)
    z = jnp.maximum(z[:, 0], z[:, 1])
    xf = z.astype(jnp.bfloat16).reshape(BT, 2048)

    xf = xf * m1_ref[...]
    h = jnp.dot(xf, w1_ref[...], preferred_element_type=jnp.float32)
    h = jnp.maximum(h + b1_ref[...], 0.0)
    h = h.astype(jnp.bfloat16) * m2_ref[...]
    y = jnp.dot(h, w2_ref[...], preferred_element_type=jnp.float32)
    o_ref[...] = y + b2_ref[...]


def _packed_w(w9, Cin, Cout, P):
    Lin, Lout = P * Cin, P * Cout
    out = jnp.zeros((9, Lin, Lout), jnp.bfloat16)
    for dy in range(3):
        for g in (-1, 0, 1):
            Wm = jnp.zeros((P, Cin, P, Cout), jnp.bfloat16)
            for q in range(P):
                for o in range(P):
                    dx = q - o + P * g
                    if -1 <= dx <= 1:
                        Wm = Wm.at[q, :, o, :].set(w9[dy * 3 + dx + 1])
            out = out.at[3 * dy + g + 1].set(Wm.reshape(Lin, Lout))
    return out


def _bn_conv_packed_kernel(y_ref, s_ref, t_ref, w_ref, b_ref,
                           o_ref, st_ref, zp_ref, *, BT, Hi, pool, Cin,
                           Cout, P):
    Pl = 2 * P if pool else P
    M2 = (BT * Hi * Hi) // Pl
    z = y_ref[...].astype(jnp.float32) * s_ref[...] + t_ref[...]
    H = Hi // 2 if pool else Hi
    Wg = H // P
    if pool:
        parts = [jnp.maximum(z[:, (2 * i) * Cin:(2 * i + 1) * Cin],
                             z[:, (2 * i + 1) * Cin:(2 * i + 2) * Cin])
                 for i in range(Pl // 2)]
        half = parts[0] if len(parts) == 1 else jnp.concatenate(
            parts, axis=1)
        v = half.reshape(BT * H, 2 * Wg, P * Cin)
        z = jnp.maximum(v[:, :Wg, :], v[:, Wg:, :])
        x_p = z.astype(jnp.bfloat16).reshape(BT, H, Wg, P * Cin)
    else:
        x_p = z.astype(jnp.bfloat16).reshape(BT, H, Wg, P * Cin)

    Lin = P * Cin
    M = BT * H * Wg
    zp_ref[...] = jnp.zeros_like(zp_ref)
    zp_ref[:, 1:H + 1, 1:Wg + 1, :] = x_p

    Lout = P * Cout
    acc = jnp.zeros((M, Lout), jnp.float32)
    for dy in range(3):
        for g in (-1, 0, 1):
            A = zp_ref[:, dy:dy + H, 1 + g:1 + g + Wg, :].reshape(M, Lin)
            acc = acc + jnp.dot(A, w_ref[3 * dy + g + 1],
                                preferred_element_type=jnp.float32)
    y = jnp.maximum(acc + b_ref[...], 0.0)
    o_ref[...] = y.astype(jnp.bfloat16)
    st_ref[0, 0:1, :] = jnp.sum(y, axis=0, keepdims=True)
    st_ref[0, 1:2, :] = jnp.sum(y * y, axis=0, keepdims=True)


def _affine_params(st, gamma, beta, Np, Pl):
    C = gamma.shape[0]
    sums = st.sum(axis=0)
    if sums.shape[-1] != C:
        sums = sums.reshape(2, sums.shape[-1] // C, C).sum(axis=1)
    mean = sums[0] / Np
    var = jnp.maximum(sums[1] / Np - mean * mean, 0.0)
    scale = gamma * jax.lax.rsqrt(var + _EPS)
    shift = beta - mean * scale
    return (jnp.tile(scale, Pl).reshape(1, Pl * C),
            jnp.tile(shift, Pl).reshape(1, Pl * C))


def _conv_stage_packed(yprev, st_prev, gamma, beta, w9, b, *,
                       BT, Hi, pool, Cin, Cout, P):
    H = Hi // 2 if pool else Hi
    Rp = Hi * Hi
    _B = yprev.shape[0] // Rp
    T = _B // BT
    Wg = H // P
    Pl = 2 * P if pool else P
    Lin, Lout = P * Cin, P * Cout
    wp = _packed_w(w9, Cin, Cout, P)
    s, t = _affine_params(st_prev, gamma, beta, _B * Rp, Pl)
    yp = yprev.reshape(_B * Rp // Pl, Pl * Cin)
    bp = jnp.tile(b.reshape(Cout), P).reshape(1, Lout)
    M = _B * H * Wg
    y, st = pl.pallas_call(
        functools.partial(_bn_conv_packed_kernel, BT=BT, Hi=Hi, pool=pool,
                          Cin=Cin, Cout=Cout, P=P),
        out_shape=(jax.ShapeDtypeStruct((M, Lout), jnp.bfloat16),
                   jax.ShapeDtypeStruct((T, 2, Lout), jnp.float32)),
        grid=(T,),
        in_specs=[
            pl.BlockSpec((BT * Rp // Pl, Pl * Cin), lambda tt: (tt, 0)),
            pl.BlockSpec((1, Pl * Cin), lambda tt: (0, 0)),
            pl.BlockSpec((1, Pl * Cin), lambda tt: (0, 0)),
            pl.BlockSpec((9, Lin, Lout), lambda tt: (0, 0, 0)),
            pl.BlockSpec((1, Lout), lambda tt: (0, 0)),
        ],
        out_specs=(
            pl.BlockSpec((BT * H * Wg, Lout), lambda tt: (tt, 0)),
            pl.BlockSpec((1, 2, Lout), lambda tt: (tt, 0, 0)),
        ),
        scratch_shapes=[pltpu.VMEM((BT, H + 2, Wg + 2, Lin),
                                   jnp.bfloat16)],
        compiler_params=pltpu.CompilerParams(
            dimension_semantics=("parallel",),
            vmem_limit_bytes=_VMEM_LIMIT),
    )(yp, s, t, wp, bp)
    return y.reshape(_B * H * H, Cout), st


def _conv_stage(yprev, st_prev, gamma, beta, w9, b, *,
                BT, Hi, pool, Cin, Cout):
    H = Hi // 2 if pool else Hi
    R = H * H
    Rp = Hi * Hi
    _B = yprev.shape[0] // Rp
    T = _B // BT
    Tp = st_prev.shape[0]
    use_col = Cin < 128
    if use_col:
        w_in = w9.reshape(9 * Cin, Cout)
        w_spec = pl.BlockSpec((9 * Cin, Cout), lambda t: (0, 0))
        scratch = [pltpu.VMEM((BT, H + 2, H + 2, Cin), jnp.bfloat16),
                   pltpu.VMEM((BT * R, 9 * Cin), jnp.bfloat16)]
    else:
        w_in = w9
        w_spec = pl.BlockSpec((9, Cin, Cout), lambda t: (0, 0, 0))
        scratch = [pltpu.VMEM((BT, H + 2, H + 2, Cin), jnp.bfloat16)]
    y, st = pl.pallas_call(
        functools.partial(_bn_conv_kernel, BT=BT, Hi=Hi, pool=pool,
                          Cin=Cin, Cout=Cout, Np=_B * Rp),
        out_shape=(jax.ShapeDtypeStruct((_B * R, Cout), jnp.bfloat16),
                   jax.ShapeDtypeStruct((T, 2, Cout), jnp.float32)),
        grid=(T,),
        in_specs=[
            pl.BlockSpec((BT * Rp, Cin), lambda t: (t, 0)),
            pl.BlockSpec((Tp, 2, Cin), lambda t: (0, 0, 0)),
            pl.BlockSpec((1, Cin), lambda t: (0, 0)),
            pl.BlockSpec((1, Cin), lambda t: (0, 0)),
            w_spec,
            pl.BlockSpec((1, Cout), lambda t: (0, 0)),
        ],
        out_specs=(
            pl.BlockSpec((BT * R, Cout), lambda t: (t, 0)),
            pl.BlockSpec((1, 2, Cout), lambda t: (t, 0, 0)),
        ),
        scratch_shapes=scratch,
        compiler_params=pltpu.CompilerParams(
            dimension_semantics=("parallel",),
            vmem_limit_bytes=_VMEM_LIMIT),
    )(yprev, st_prev, gamma.reshape(1, Cin), beta.reshape(1, Cin), w_in, b)
    return y, st


def kernel(b1c1_w9, b1c1_b, b1c1_gamma, b1c1_beta,
           b1c2_w9, b1c2_b, b1c2_gamma, b1c2_beta,
           b2c1_w9, b2c1_b, b2c1_gamma, b2c1_beta,
           b2c2_w9, b2c2_b, b2c2_gamma, b2c2_beta,
           b3c1_w9, b3c1_b, b3c1_gamma, b3c1_beta,
           b3c2_w9, b3c2_b, b3c2_gamma, b3c2_beta,
           fc1_w, fc1_b, fc2_w, fc2_b,
           x, m1, m2):
    B = x.shape[0]
    x2 = x.reshape(B * 3, 1024)
    w72 = jnp.zeros((9, 8, 32), jnp.bfloat16).at[:, :3, :].set(
        b1c1_w9).reshape(72, 32)

    BT1 = 8
    T1 = B // BT1
    y1, st1 = pl.pallas_call(
        functools.partial(_conv1_kernel, BT=BT1),
        out_shape=(jax.ShapeDtypeStruct((B * 1024, 32), jnp.bfloat16),
                   jax.ShapeDtypeStruct((T1, 2, 32), jnp.float32)),
        grid=(T1,),
        in_specs=[
            pl.BlockSpec((BT1 * 3, 1024), lambda t: (t, 0)),
            pl.BlockSpec((72, 32), lambda t: (0, 0)),
            pl.BlockSpec((1, 32), lambda t: (0, 0)),
        ],
        out_specs=(
            pl.BlockSpec((BT1 * 1024, 32), lambda t: (t, 0)),
            pl.BlockSpec((1, 2, 32), lambda t: (t, 0, 0)),
        ),
        scratch_shapes=[pltpu.VMEM((BT1, 34, 34, 8), jnp.bfloat16),
                        pltpu.VMEM((BT1 * 1024, 72), jnp.bfloat16)],
        compiler_params=pltpu.CompilerParams(
            dimension_semantics=("parallel",),
            vmem_limit_bytes=_VMEM_LIMIT),
    )(x2, w72, b1c1_b)

    y2, st2 = _conv_stage_packed(y1, st1, b1c1_gamma, b1c1_beta, b1c2_w9,
                                 b1c2_b, BT=8, Hi=32, pool=False, Cin=32,
                                 Cout=32, P=4)
    y3, st3 = _conv_stage_packed(y2, st2, b1c2_gamma, b1c2_beta, b2c1_w9,
                                 b2c1_b, BT=8, Hi=32, pool=True, Cin=32,
                                 Cout=64, P=2)
    y4, st4 = _conv_stage_packed(y3, st3, b2c1_gamma, b2c1_beta, b2c2_w9,
                                 b2c2_b, BT=16, Hi=16, pool=False, Cin=64,
                                 Cout=64, P=2)
    st4s = st4.reshape(st4.shape[0], 2, 2, 64).sum(axis=2)
    y5, st5 = _conv_stage(y4, st4s, b2c2_gamma, b2c2_beta, b3c1_w9, b3c1_b,
                          BT=16, Hi=16, pool=True, Cin=64, Cout=128)
    y6, st6 = _conv_stage(y5, st5, b3c1_gamma, b3c1_beta, b3c2_w9, b3c2_b,
                          BT=32, Hi=8, pool=False, Cin=128, Cout=128)

    BT7 = 128
    T6 = st6.shape[0]
    Dpad = fc2_w.shape[1]
    logits = pl.pallas_call(
        functools.partial(_head_kernel, BT=BT7, Np=B * 64),
        out_shape=jax.ShapeDtypeStruct((B, Dpad), jnp.float32),
        grid=(B // BT7,),
        in_specs=[
            pl.BlockSpec((BT7 * 64, 128), lambda t: (t, 0)),
            pl.BlockSpec((T6, 2, 128), lambda t: (0, 0, 0)),
            pl.BlockSpec((1, 128), lambda t: (0, 0)),
            pl.BlockSpec((1, 128), lambda t: (0, 0)),
            pl.BlockSpec((BT7, 2048), lambda t: (t, 0)),
            pl.BlockSpec((2048, 1024), lambda t: (0, 0)),
            pl.BlockSpec((1, 1024), lambda t: (0, 0)),
            pl.BlockSpec((BT7, 1024), lambda t: (t, 0)),
            pl.BlockSpec((1024, Dpad), lambda t: (0, 0)),
            pl.BlockSpec((1, Dpad), lambda t: (0, 0)),
        ],
        out_specs=pl.BlockSpec((BT7, Dpad), lambda t: (t, 0)),
        compiler_params=pltpu.CompilerParams(
            dimension_semantics=("parallel",),
            vmem_limit_bytes=_VMEM_LIMIT),
    )(y6, st6, b3c2_gamma.reshape(1, 128), b3c2_beta.reshape(1, 128),
      m1, fc1_w, fc1_b, m2, fc2_w, fc2_b)
    return logits[:, :10]


# speedup vs baseline: 3.1199x; 1.0671x over previous
import functools

import jax
import jax.numpy as jnp
from jax.experimental import pallas as pl
from jax.experimental.pallas import tpu as pltpu

_EPS = 1e-5
_VMEM_LIMIT = 48 * 1024 * 1024


def _tile_stats(st_ref, y):
    st_ref[0, 0:1, :] = jnp.sum(y, axis=0, keepdims=True)
    st_ref[0, 1:2, :] = jnp.sum(y * y, axis=0, keepdims=True)


def _conv1_kernel(x_ref, w_ref, b_ref, y_ref, st_ref, xp_ref, col_ref, *,
                  BT):
    H = W = 32
    R = H * W
    xt = jnp.transpose(x_ref[...].reshape(BT, 3, R), (0, 2, 1))
    xi = xt.astype(jnp.bfloat16).reshape(BT, H, W, 3)
    xp_ref[...] = jnp.zeros_like(xp_ref)
    xp_ref[:, 1:H + 1, 1:W + 1, :3] = xi
    for k in range(9):
        ky, kx = divmod(k, 3)
        col_ref[:, 8 * k:8 * k + 8] = (
            xp_ref[:, ky:ky + H, kx:kx + W, :].reshape(BT * R, 8))
    acc = jnp.dot(col_ref[...], w_ref[...],
                  preferred_element_type=jnp.float32)
    y = jnp.maximum(acc + b_ref[...], 0.0)
    y_ref[...] = y.astype(jnp.bfloat16)
    _tile_stats(st_ref, y)


def _bn_conv_kernel(y_ref, st_in_ref, g_ref, be_ref, w_ref, b_ref,
                    o_ref, st_ref, xp_ref, col_ref=None, *, BT, Hi, pool,
                    Cin, Cout, Np):
    sums = jnp.sum(st_in_ref[...], axis=0)
    mean = sums[0:1, :] * (1.0 / Np)
    var = jnp.maximum(sums[1:2, :] * (1.0 / Np) - mean * mean, 0.0)
    scale = g_ref[...] * jax.lax.rsqrt(var + _EPS)
    shift = be_ref[...] - mean * scale

    z = y_ref[...].astype(jnp.float32) * scale + shift
    H = Hi // 2 if pool else Hi
    if pool:
        z = z.reshape(BT * Hi * H, 2, Cin)
        z = jnp.maximum(z[:, 0, :], z[:, 1, :])
        z = z.reshape(BT * H, 2, H, Cin)
        z = jnp.maximum(z[:, 0], z[:, 1])
    x = z.astype(jnp.bfloat16).reshape(BT, H, H, Cin)

    xp_ref[...] = jnp.zeros_like(xp_ref)
    xp_ref[:, 1:H + 1, 1:H + 1, :] = x

    R = H * H
    if col_ref is not None:
        for k in range(9):
            ky, kx = divmod(k, 3)
            col_ref[:, Cin * k:Cin * (k + 1)] = (
                xp_ref[:, ky:ky + H, kx:kx + H, :].reshape(BT * R, Cin))
        acc = jnp.dot(col_ref[...], w_ref[...],
                      preferred_element_type=jnp.float32)
    else:
        acc = jnp.zeros((BT * R, Cout), jnp.float32)
        for k in range(9):
            ky, kx = divmod(k, 3)
            tap = xp_ref[:, ky:ky + H, kx:kx + H, :].reshape(BT * R, Cin)
            acc = acc + jnp.dot(tap, w_ref[k],
                                preferred_element_type=jnp.float32)
    y = jnp.maximum(acc + b_ref[...], 0.0)
    o_ref[...] = y.astype(jnp.bfloat16)
    _tile_stats(st_ref, y)


def _head_kernel(y_ref, st_in_ref, g_ref, be_ref, m1_ref, w1_ref, b1_ref,
                 m2_ref, w2_ref, b2_ref, o_ref, *, BT, Np):
    sums = jnp.sum(st_in_ref[...], axis=0)
    mean = sums[0:1, :] * (1.0 / Np)
    var = jnp.maximum(sums[1:2, :] * (1.0 / Np) - mean * mean, 0.0)
    scale = g_ref[...] * jax.lax.rsqrt(var + _EPS)
    shift = be_ref[...] - mean * scale

    z = y_ref[...].astype(jnp.float32) * scale + shift
    z = z.reshape(BT * 8 * 4, 2, 128)
    z = jnp.maximum(z[:, 0, :], z[:, 1, :])
    z = z.reshape(BT * 4, 2, 4, 128)
    z = jnp.maximum(z[:, 0], z[:, 1])
    xf = z.astype(jnp.bfloat16).reshape(BT, 2048)

    xf = xf * m1_ref[...]
    h = jnp.dot(xf, w1_ref[...], preferred_element_type=jnp.float32)
    h = jnp.maximum(h + b1_ref[...], 0.0)
    h = h.astype(jnp.bfloat16) * m2_ref[...]
    y = jnp.dot(h, w2_ref[...], preferred_element_type=jnp.float32)
    o_ref[...] = y + b2_ref[...]


def _packed_w(w9, Cin, Cout, P):
    Lin, Lout = P * Cin, P * Cout
    out = jnp.zeros((9, Lin, Lout), jnp.bfloat16)
    for dy in range(3):
        for g in (-1, 0, 1):
            Wm = jnp.zeros((P, Cin, P, Cout), jnp.bfloat16)
            for q in range(P):
                for o in range(P):
                    dx = q - o + P * g
                    if -1 <= dx <= 1:
                        Wm = Wm.at[q, :, o, :].set(w9[dy * 3 + dx + 1])
            out = out.at[3 * dy + g + 1].set(Wm.reshape(Lin, Lout))
    return out


def _bn_conv_packed_kernel(y_ref, s_ref, t_ref, w_ref, b_ref,
                           o_ref, st_ref, zp_ref, *, BT, Hi, pool, Cin,
                           Cout, P):
    Pl = 2 * P if pool else P
    M2 = (BT * Hi * Hi) // Pl
    z = y_ref[...].astype(jnp.float32) * s_ref[...] + t_ref[...]
    H = Hi // 2 if pool else Hi
    Wg = H // P
    if pool:
        parts = [jnp.maximum(z[:, (2 * i) * Cin:(2 * i + 1) * Cin],
                             z[:, (2 * i + 1) * Cin:(2 * i + 2) * Cin])
                 for i in range(Pl // 2)]
        half = parts[0] if len(parts) == 1 else jnp.concatenate(
            parts, axis=1)
        v = half.reshape(BT * H, 2 * Wg, P * Cin)
        z = jnp.maximum(v[:, :Wg, :], v[:, Wg:, :])
        x_p = z.astype(jnp.bfloat16).reshape(BT, H, Wg, P * Cin)
    else:
        x_p = z.astype(jnp.bfloat16).reshape(BT, H, Wg, P * Cin)

    Lin = P * Cin
    M = BT * H * Wg
    zp_ref[...] = jnp.zeros_like(zp_ref)
    zp_ref[:, 1:H + 1, 1:Wg + 1, :] = x_p

    Lout = P * Cout
    acc = jnp.zeros((M, Lout), jnp.float32)
    for dy in range(3):
        for g in (-1, 0, 1):
            A = zp_ref[:, dy:dy + H, 1 + g:1 + g + Wg, :].reshape(M, Lin)
            acc = acc + jnp.dot(A, w_ref[3 * dy + g + 1],
                                preferred_element_type=jnp.float32)
    y = jnp.maximum(acc + b_ref[...], 0.0)
    o_ref[...] = y.astype(jnp.bfloat16)
    st_ref[0, 0:1, :] = jnp.sum(y, axis=0, keepdims=True)
    st_ref[0, 1:2, :] = jnp.sum(y * y, axis=0, keepdims=True)


def _affine_params(st, gamma, beta, Np, Pl):
    C = gamma.shape[0]
    sums = st.sum(axis=0)
    if sums.shape[-1] != C:
        sums = sums.reshape(2, sums.shape[-1] // C, C).sum(axis=1)
    mean = sums[0] / Np
    var = jnp.maximum(sums[1] / Np - mean * mean, 0.0)
    scale = gamma * jax.lax.rsqrt(var + _EPS)
    shift = beta - mean * scale
    return (jnp.tile(scale, Pl).reshape(1, Pl * C),
            jnp.tile(shift, Pl).reshape(1, Pl * C))


def _conv_stage_packed(yprev, st_prev, gamma, beta, w9, b, *,
                       BT, Hi, pool, Cin, Cout, P):
    H = Hi // 2 if pool else Hi
    Rp = Hi * Hi
    _B = yprev.shape[0] // Rp
    T = _B // BT
    Wg = H // P
    Pl = 2 * P if pool else P
    Lin, Lout = P * Cin, P * Cout
    wp = _packed_w(w9, Cin, Cout, P)
    s, t = _affine_params(st_prev, gamma, beta, _B * Rp, Pl)
    yp = yprev.reshape(_B * Rp // Pl, Pl * Cin)
    bp = jnp.tile(b.reshape(Cout), P).reshape(1, Lout)
    M = _B * H * Wg
    y, st = pl.pallas_call(
        functools.partial(_bn_conv_packed_kernel, BT=BT, Hi=Hi, pool=pool,
                          Cin=Cin, Cout=Cout, P=P),
        out_shape=(jax.ShapeDtypeStruct((M, Lout), jnp.bfloat16),
                   jax.ShapeDtypeStruct((T, 2, Lout), jnp.float32)),
        grid=(T,),
        in_specs=[
            pl.BlockSpec((BT * Rp // Pl, Pl * Cin), lambda tt: (tt, 0)),
            pl.BlockSpec((1, Pl * Cin), lambda tt: (0, 0)),
            pl.BlockSpec((1, Pl * Cin), lambda tt: (0, 0)),
            pl.BlockSpec((9, Lin, Lout), lambda tt: (0, 0, 0)),
            pl.BlockSpec((1, Lout), lambda tt: (0, 0)),
        ],
        out_specs=(
            pl.BlockSpec((BT * H * Wg, Lout), lambda tt: (tt, 0)),
            pl.BlockSpec((1, 2, Lout), lambda tt: (tt, 0, 0)),
        ),
        scratch_shapes=[pltpu.VMEM((BT, H + 2, Wg + 2, Lin),
                                   jnp.bfloat16)],
        compiler_params=pltpu.CompilerParams(
            dimension_semantics=("parallel",),
            vmem_limit_bytes=_VMEM_LIMIT),
    )(yp, s, t, wp, bp)
    return y.reshape(_B * H * H, Cout), st


def _conv_stage(yprev, st_prev, gamma, beta, w9, b, *,
                BT, Hi, pool, Cin, Cout):
    H = Hi // 2 if pool else Hi
    R = H * H
    Rp = Hi * Hi
    _B = yprev.shape[0] // Rp
    T = _B // BT
    Tp = st_prev.shape[0]
    use_col = Cin < 128
    if use_col:
        w_in = w9.reshape(9 * Cin, Cout)
        w_spec = pl.BlockSpec((9 * Cin, Cout), lambda t: (0, 0))
        scratch = [pltpu.VMEM((BT, H + 2, H + 2, Cin), jnp.bfloat16),
                   pltpu.VMEM((BT * R, 9 * Cin), jnp.bfloat16)]
    else:
        w_in = w9
        w_spec = pl.BlockSpec((9, Cin, Cout), lambda t: (0, 0, 0))
        scratch = [pltpu.VMEM((BT, H + 2, H + 2, Cin), jnp.bfloat16)]
    y, st = pl.pallas_call(
        functools.partial(_bn_conv_kernel, BT=BT, Hi=Hi, pool=pool,
                          Cin=Cin, Cout=Cout, Np=_B * Rp),
        out_shape=(jax.ShapeDtypeStruct((_B * R, Cout), jnp.bfloat16),
                   jax.ShapeDtypeStruct((T, 2, Cout), jnp.float32)),
        grid=(T,),
        in_specs=[
            pl.BlockSpec((BT * Rp, Cin), lambda t: (t, 0)),
            pl.BlockSpec((Tp, 2, Cin), lambda t: (0, 0, 0)),
            pl.BlockSpec((1, Cin), lambda t: (0, 0)),
            pl.BlockSpec((1, Cin), lambda t: (0, 0)),
            w_spec,
            pl.BlockSpec((1, Cout), lambda t: (0, 0)),
        ],
        out_specs=(
            pl.BlockSpec((BT * R, Cout), lambda t: (t, 0)),
            pl.BlockSpec((1, 2, Cout), lambda t: (t, 0, 0)),
        ),
        scratch_shapes=scratch,
        compiler_params=pltpu.CompilerParams(
            dimension_semantics=("parallel",),
            vmem_limit_bytes=_VMEM_LIMIT),
    )(yprev, st_prev, gamma.reshape(1, Cin), beta.reshape(1, Cin), w_in, b)
    return y, st


def kernel(b1c1_w9, b1c1_b, b1c1_gamma, b1c1_beta,
           b1c2_w9, b1c2_b, b1c2_gamma, b1c2_beta,
           b2c1_w9, b2c1_b, b2c1_gamma, b2c1_beta,
           b2c2_w9, b2c2_b, b2c2_gamma, b2c2_beta,
           b3c1_w9, b3c1_b, b3c1_gamma, b3c1_beta,
           b3c2_w9, b3c2_b, b3c2_gamma, b3c2_beta,
           fc1_w, fc1_b, fc2_w, fc2_b,
           x, m1, m2):
    B = x.shape[0]
    x2 = x.reshape(B * 3, 1024)
    w72 = jnp.zeros((9, 8, 32), jnp.bfloat16).at[:, :3, :].set(
        b1c1_w9).reshape(72, 32)

    BT1 = 8
    T1 = B // BT1
    y1, st1 = pl.pallas_call(
        functools.partial(_conv1_kernel, BT=BT1),
        out_shape=(jax.ShapeDtypeStruct((B * 1024, 32), jnp.bfloat16),
                   jax.ShapeDtypeStruct((T1, 2, 32), jnp.float32)),
        grid=(T1,),
        in_specs=[
            pl.BlockSpec((BT1 * 3, 1024), lambda t: (t, 0)),
            pl.BlockSpec((72, 32), lambda t: (0, 0)),
            pl.BlockSpec((1, 32), lambda t: (0, 0)),
        ],
        out_specs=(
            pl.BlockSpec((BT1 * 1024, 32), lambda t: (t, 0)),
            pl.BlockSpec((1, 2, 32), lambda t: (t, 0, 0)),
        ),
        scratch_shapes=[pltpu.VMEM((BT1, 34, 34, 8), jnp.bfloat16),
                        pltpu.VMEM((BT1 * 1024, 72), jnp.bfloat16)],
        compiler_params=pltpu.CompilerParams(
            dimension_semantics=("parallel",),
            vmem_limit_bytes=_VMEM_LIMIT),
    )(x2, w72, b1c1_b)

    y2, st2 = _conv_stage_packed(y1, st1, b1c1_gamma, b1c1_beta, b1c2_w9,
                                 b1c2_b, BT=16, Hi=32, pool=False, Cin=32,
                                 Cout=32, P=4)
    y3, st3 = _conv_stage_packed(y2, st2, b1c2_gamma, b1c2_beta, b2c1_w9,
                                 b2c1_b, BT=16, Hi=32, pool=True, Cin=32,
                                 Cout=64, P=2)
    y4, st4 = _conv_stage_packed(y3, st3, b2c1_gamma, b2c1_beta, b2c2_w9,
                                 b2c2_b, BT=32, Hi=16, pool=False, Cin=64,
                                 Cout=64, P=2)
    y5, st5 = _conv_stage_packed(y4, st4, b2c2_gamma, b2c2_beta, b3c1_w9,
                                 b3c1_b, BT=16, Hi=16, pool=True, Cin=64,
                                 Cout=128, P=1)
    y6, st6 = _conv_stage_packed(y5, st5, b3c1_gamma, b3c1_beta, b3c2_w9,
                                 b3c2_b, BT=32, Hi=8, pool=False, Cin=128,
                                 Cout=128, P=1)

    BT7 = 128
    T6 = st6.shape[0]
    Dpad = fc2_w.shape[1]
    logits = pl.pallas_call(
        functools.partial(_head_kernel, BT=BT7, Np=B * 64),
        out_shape=jax.ShapeDtypeStruct((B, Dpad), jnp.float32),
        grid=(B // BT7,),
        in_specs=[
            pl.BlockSpec((BT7 * 64, 128), lambda t: (t, 0)),
            pl.BlockSpec((T6, 2, 128), lambda t: (0, 0, 0)),
            pl.BlockSpec((1, 128), lambda t: (0, 0)),
            pl.BlockSpec((1, 128), lambda t: (0, 0)),
            pl.BlockSpec((BT7, 2048), lambda t: (t, 0)),
            pl.BlockSpec((2048, 1024), lambda t: (0, 0)),
            pl.BlockSpec((1, 1024), lambda t: (0, 0)),
            pl.BlockSpec((BT7, 1024), lambda t: (t, 0)),
            pl.BlockSpec((1024, Dpad), lambda t: (0, 0)),
            pl.BlockSpec((1, Dpad), lambda t: (0, 0)),
        ],
        out_specs=pl.BlockSpec((BT7, Dpad), lambda t: (t, 0)),
        compiler_params=pltpu.CompilerParams(
            dimension_semantics=("parallel",),
            vmem_limit_bytes=_VMEM_LIMIT),
    )(y6, st6, b3c2_gamma.reshape(1, 128), b3c2_beta.reshape(1, 128),
      m1, fc1_w, fc1_b, m2, fc2_w, fc2_b)
    return logits[:, :10]
